# Initial kernel scaffold; baseline (speedup 1.0000x reference)
import jax
import jax.numpy as jnp
from jax.experimental import pallas as pl


def kernel(timestamps, keypoints, keypoint_confidence, trial_lengths, idx):
    raise NotImplementedError("write your pallas kernel here")


# speedup vs baseline: 17.0959x; 17.0959x over previous
import jax
import jax.numpy as jnp
from jax import lax
from jax.experimental import pallas as pl
from jax.experimental.pallas import tpu as pltpu
from jax.experimental.pallas import tpu_sc as plsc

NUM_TRIALS = 16
MAX_LEN = 4096
CAMERAS = 6
KPTS = 25
SAMPLE_LENGTH = 1024
BATCH = 64

NC = 2
NS = 16
NW = NC * NS
B_PER_W = BATCH // NW

KP_D = KPTS * 2
FUSE_D = 80
ROWS = SAMPLE_LENGTH
CHUNK = 128
NCHUNK = ROWS // CHUNK


def _splat(vec, idxvec):
    dnums = lax.GatherDimensionNumbers(
        offset_dims=(), collapsed_slice_dims=(0,), start_index_map=(0,))
    return lax.gather(vec, idxvec.reshape(16, 1), dnums, (1,),
                      mode=lax.GatherScatterMode.PROMISE_IN_BOUNDS)


def _sc_body(fused_hbm, len_hbm, idx_hbm,
             ii_out, f_out,
             idx_v, len_v, gidx_v, buf_v, ii_buf, sem):
    c = lax.axis_index("c")
    s = lax.axis_index("s")
    wid = s * NC + c

    pltpu.sync_copy(idx_hbm, idx_v)
    pltpu.sync_copy(len_hbm, len_v)

    @pl.when(wid == 0)
    def _():
        for k in range(BATCH // 16):
            ii_buf[pl.ds(k * 16, 16)] = idx_v[pl.ds(k * 16, 16)] & (
                NUM_TRIALS - 1)
        pltpu.sync_copy(ii_buf, ii_out)

    len16 = len_v[...]
    iot = lax.iota(jnp.int32, 16)

    for r in range(B_PER_W):
        b = wid * B_PER_W + r
        win = idx_v[pl.ds(((b >> 4) << 4), 16)]
        iv_v = _splat(win, iot * 0 + (b & 15))
        start_v = iv_v >> 4
        ii_v = iv_v & (NUM_TRIALS - 1)
        lv_v = _splat(len16, ii_v)
        nv_v = (lv_v + (SAMPLE_LENGTH - 1)) >> 10
        base_v = ii_v * (CAMERAS * MAX_LEN)

        for rw in range(NCHUNK):
            for k in range(CHUNK // 16):
                tv = iot + (rw * CHUNK + k * 16)
                vv = tv * nv_v + start_v
                sv = vv - jnp.where(vv >= lv_v, lv_v, 0)
                rb = base_v + sv
                for cam in range(CAMERAS):
                    gidx_v[cam * NCHUNK + rw, pl.ds(k * 16, 16)] = (
                        rb + cam * MAX_LEN)

        def cam_body(cam, _):
            cps = [
                pltpu.async_copy(
                    fused_hbm.at[gidx_v.at[cam * NCHUNK + j]],
                    buf_v.at[pl.ds(j * CHUNK, CHUNK)], sem)
                for j in range(NCHUNK)
            ]
            for cp in cps:
                cp.wait()
            out_row = (b * CAMERAS + cam) * ROWS
            pltpu.sync_copy(buf_v, f_out.at[pl.ds(out_row, ROWS)])
            return 0

        lax.fori_loop(0, CAMERAS, cam_body, 0)


@jax.jit
def kernel(timestamps, keypoints, keypoint_confidence, trial_lengths, idx):
    kp2 = keypoints.reshape(NUM_TRIALS, CAMERAS, MAX_LEN, KP_D)
    conf2 = keypoint_confidence.astype(jnp.float32)
    ts_b = jnp.broadcast_to(
        timestamps.astype(jnp.float32)[:, None, :, None],
        (NUM_TRIALS, CAMERAS, MAX_LEN, 1))
    pad = jnp.zeros((NUM_TRIALS, CAMERAS, MAX_LEN, FUSE_D - KP_D - KPTS - 1),
                    jnp.float32)
    fused = jnp.concatenate([kp2, conf2, ts_b, pad], axis=-1).reshape(
        NUM_TRIALS * CAMERAS * MAX_LEN, FUSE_D)
    lens = trial_lengths.astype(jnp.int32)
    idx32 = idx.astype(jnp.int32)

    mesh = plsc.VectorSubcoreMesh(core_axis_name="c", subcore_axis_name="s")
    run = pl.kernel(
        _sc_body,
        out_type=[
            jax.ShapeDtypeStruct((BATCH,), jnp.int32),
            jax.ShapeDtypeStruct((BATCH * CAMERAS * SAMPLE_LENGTH, FUSE_D),
                                 jnp.float32),
        ],
        mesh=mesh,
        compiler_params=pltpu.CompilerParams(use_tc_tiling_on_sc=False),
        scratch_types=[
            pltpu.VMEM((BATCH,), jnp.int32),
            pltpu.VMEM((NUM_TRIALS,), jnp.int32),
            pltpu.VMEM((CAMERAS * NCHUNK, CHUNK), jnp.int32),
            pltpu.VMEM((ROWS, FUSE_D), jnp.float32),
            pltpu.VMEM((BATCH,), jnp.int32),
            pltpu.SemaphoreType.DMA,
        ],
    )
    ii, fo = run(fused, lens, idx32)
    kp_s = fo[:, :KP_D].reshape(BATCH, CAMERAS, SAMPLE_LENGTH, KPTS, 2)
    conf_s = fo[:, KP_D:KP_D + KPTS].reshape(BATCH, CAMERAS, SAMPLE_LENGTH,
                                             KPTS)
    ts_s = fo.reshape(BATCH, CAMERAS, SAMPLE_LENGTH, FUSE_D)[:, 0, :,
                                                             KP_D + KPTS]
    return ((ii, ts_s), (kp_s, conf_s))


# speedup vs baseline: 17.0961x; 1.0000x over previous
import jax
import jax.numpy as jnp
from jax import lax
from jax.experimental import pallas as pl
from jax.experimental.pallas import tpu as pltpu
from jax.experimental.pallas import tpu_sc as plsc

NUM_TRIALS = 16
MAX_LEN = 4096
CAMERAS = 6
KPTS = 25
SAMPLE_LENGTH = 1024
BATCH = 64

NC = 2
NS = 16
NW = NC * NS
B_PER_W = BATCH // NW

KP_D = KPTS * 2
FUSE_D = 80
ROWS = SAMPLE_LENGTH
CHUNK = 128
NCHUNK = ROWS // CHUNK
HROWS = ROWS // 2
HCHUNK = NCHUNK // 2


def _splat(vec, idxvec):
    dnums = lax.GatherDimensionNumbers(
        offset_dims=(), collapsed_slice_dims=(0,), start_index_map=(0,))
    return lax.gather(vec, idxvec.reshape(16, 1), dnums, (1,),
                      mode=lax.GatherScatterMode.PROMISE_IN_BOUNDS)


def _sc_body(fused_hbm, len_hbm, idx_hbm,
             ii_out, f_out,
             idx_v, len_v, gidx_v, buf_v, ii_buf, sem, wsem):
    c = lax.axis_index("c")
    s = lax.axis_index("s")
    wid = s * NC + c

    pltpu.sync_copy(idx_hbm, idx_v)
    pltpu.sync_copy(len_hbm, len_v)

    @pl.when(wid == 0)
    def _():
        for k in range(BATCH // 16):
            ii_buf[pl.ds(k * 16, 16)] = idx_v[pl.ds(k * 16, 16)] & (
                NUM_TRIALS - 1)
        pltpu.sync_copy(ii_buf, ii_out)

    len16 = len_v[...]
    iot = lax.iota(jnp.int32, 16)
    wr_cps = [None, None]

    for r in range(B_PER_W):
        b = wid * B_PER_W + r
        win = idx_v[pl.ds(((b >> 4) << 4), 16)]
        iv_v = _splat(win, iot * 0 + (b & 15))
        start_v = iv_v >> 4
        ii_v = iv_v & (NUM_TRIALS - 1)
        lv_v = _splat(len16, ii_v)
        nv_v = (lv_v + (SAMPLE_LENGTH - 1)) >> 10
        base_v = ii_v * (CAMERAS * MAX_LEN)

        for rw in range(NCHUNK):
            for k in range(CHUNK // 16):
                tv = iot + (rw * CHUNK + k * 16)
                vv = tv * nv_v + start_v
                sv = vv - jnp.where(vv >= lv_v, lv_v, 0)
                rb = base_v + sv
                for cam in range(CAMERAS):
                    gidx_v[cam * NCHUNK + rw, pl.ds(k * 16, 16)] = (
                        rb + cam * MAX_LEN)

        for cam in range(CAMERAS):
            for h in range(2):
                p = (cam * 2 + h) & 1
                cps = [
                    pltpu.async_copy(
                        fused_hbm.at[gidx_v.at[cam * NCHUNK + h * HCHUNK + j]],
                        buf_v.at[p, pl.ds(j * CHUNK, CHUNK)], sem)
                    for j in range(HCHUNK)
                ]
                for cp in cps:
                    cp.wait()
                if wr_cps[p] is not None:
                    wr_cps[p].wait()
                out_row = (b * CAMERAS + cam) * ROWS + h * HROWS
                wr_cps[p] = pltpu.async_copy(
                    buf_v.at[p], f_out.at[pl.ds(out_row, HROWS)], wsem)

    for p in range(2):
        if wr_cps[p] is not None:
            wr_cps[p].wait()


@jax.jit
def kernel(timestamps, keypoints, keypoint_confidence, trial_lengths, idx):
    kp2 = keypoints.reshape(NUM_TRIALS, CAMERAS, MAX_LEN, KP_D)
    conf2 = keypoint_confidence.astype(jnp.float32)
    ts_b = jnp.broadcast_to(
        timestamps.astype(jnp.float32)[:, None, :, None],
        (NUM_TRIALS, CAMERAS, MAX_LEN, 1))
    pad = jnp.zeros((NUM_TRIALS, CAMERAS, MAX_LEN, FUSE_D - KP_D - KPTS - 1),
                    jnp.float32)
    fused = jnp.concatenate([kp2, conf2, ts_b, pad], axis=-1).reshape(
        NUM_TRIALS * CAMERAS * MAX_LEN, FUSE_D)
    lens = trial_lengths.astype(jnp.int32)
    idx32 = idx.astype(jnp.int32)

    mesh = plsc.VectorSubcoreMesh(core_axis_name="c", subcore_axis_name="s")
    run = pl.kernel(
        _sc_body,
        out_type=[
            jax.ShapeDtypeStruct((BATCH,), jnp.int32),
            jax.ShapeDtypeStruct((BATCH * CAMERAS * SAMPLE_LENGTH, FUSE_D),
                                 jnp.float32),
        ],
        mesh=mesh,
        compiler_params=pltpu.CompilerParams(use_tc_tiling_on_sc=False),
        scratch_types=[
            pltpu.VMEM((BATCH,), jnp.int32),
            pltpu.VMEM((NUM_TRIALS,), jnp.int32),
            pltpu.VMEM((CAMERAS * NCHUNK, CHUNK), jnp.int32),
            pltpu.VMEM((2, HROWS, FUSE_D), jnp.float32),
            pltpu.VMEM((BATCH,), jnp.int32),
            pltpu.SemaphoreType.DMA,
            pltpu.SemaphoreType.DMA,
        ],
    )
    ii, fo = run(fused, lens, idx32)
    kp_s = fo[:, :KP_D].reshape(BATCH, CAMERAS, SAMPLE_LENGTH, KPTS, 2)
    conf_s = fo[:, KP_D:KP_D + KPTS].reshape(BATCH, CAMERAS, SAMPLE_LENGTH,
                                             KPTS)
    ts_s = fo.reshape(BATCH, CAMERAS, SAMPLE_LENGTH, FUSE_D)[:, 0, :,
                                                             KP_D + KPTS]
    return ((ii, ts_s), (kp_s, conf_s))


# speedup vs baseline: 17.8376x; 1.0434x over previous
import jax
import jax.numpy as jnp
from jax import lax
from jax.experimental import pallas as pl
from jax.experimental.pallas import tpu as pltpu
from jax.experimental.pallas import tpu_sc as plsc

NUM_TRIALS = 16
MAX_LEN = 4096
CAMERAS = 6
KPTS = 25
SAMPLE_LENGTH = 1024
BATCH = 64

NC = 2
NS = 16
NW = NC * NS
B_PER_W = BATCH // NW

KP_D = KPTS * 2
FUSE_D = 80
ROWS = SAMPLE_LENGTH
CHUNK = 128
NCHUNK = ROWS // CHUNK
HROWS = ROWS // 2
HCHUNK = NCHUNK // 2


PANEL = MAX_LEN
NPANEL = NUM_TRIALS * CAMERAS


def _build_body(kp_ref, conf_ref, ts_ref, out_ref):
    kp = kp_ref[...]
    conf = conf_ref[...]
    ii = pl.program_id(0) // CAMERAS
    lane = lax.broadcasted_iota(jnp.int32, (PANEL, NUM_TRIALS), 1)
    ts = jnp.sum(jnp.where(lane == ii, ts_ref[...], 0.0), axis=1,
                 keepdims=True)
    pad = jnp.zeros((PANEL, FUSE_D - KP_D - KPTS - 1), jnp.float32)
    out_ref[...] = jnp.concatenate([kp, conf, ts, pad], axis=1)


def _split_body(f_ref, kp_ref, conf_ref):
    f = f_ref[...]
    kp_ref[...] = f[:, :KP_D]
    conf_ref[...] = f[:, KP_D:KP_D + KPTS]


def _splat(vec, idxvec):
    dnums = lax.GatherDimensionNumbers(
        offset_dims=(), collapsed_slice_dims=(0,), start_index_map=(0,))
    return lax.gather(vec, idxvec.reshape(16, 1), dnums, (1,),
                      mode=lax.GatherScatterMode.PROMISE_IN_BOUNDS)


def _sc_body(fused_hbm, len_hbm, idx_hbm,
             ii_out, f_out,
             idx_v, len_v, gidx_v, buf_v, ii_buf, sem, wsem):
    c = lax.axis_index("c")
    s = lax.axis_index("s")
    wid = s * NC + c

    pltpu.sync_copy(idx_hbm, idx_v)
    pltpu.sync_copy(len_hbm, len_v)

    @pl.when(wid == 0)
    def _():
        for k in range(BATCH // 16):
            ii_buf[pl.ds(k * 16, 16)] = idx_v[pl.ds(k * 16, 16)] & (
                NUM_TRIALS - 1)
        pltpu.sync_copy(ii_buf, ii_out)

    len16 = len_v[...]
    iot = lax.iota(jnp.int32, 16)
    wr_cps = [None, None]

    for r in range(B_PER_W):
        b = wid * B_PER_W + r
        win = idx_v[pl.ds(((b >> 4) << 4), 16)]
        iv_v = _splat(win, iot * 0 + (b & 15))
        start_v = iv_v >> 4
        ii_v = iv_v & (NUM_TRIALS - 1)
        lv_v = _splat(len16, ii_v)
        nv_v = (lv_v + (SAMPLE_LENGTH - 1)) >> 10
        base_v = ii_v * (CAMERAS * MAX_LEN)

        for rw in range(NCHUNK):
            for k in range(CHUNK // 16):
                tv = iot + (rw * CHUNK + k * 16)
                vv = tv * nv_v + start_v
                sv = vv - jnp.where(vv >= lv_v, lv_v, 0)
                rb = base_v + sv
                for cam in range(CAMERAS):
                    gidx_v[cam * NCHUNK + rw, pl.ds(k * 16, 16)] = (
                        rb + cam * MAX_LEN)

        for cam in range(CAMERAS):
            for h in range(2):
                p = (cam * 2 + h) & 1
                cps = [
                    pltpu.async_copy(
                        fused_hbm.at[gidx_v.at[cam * NCHUNK + h * HCHUNK + j]],
                        buf_v.at[p, pl.ds(j * CHUNK, CHUNK)], sem)
                    for j in range(HCHUNK)
                ]
                for cp in cps:
                    cp.wait()
                if wr_cps[p] is not None:
                    wr_cps[p].wait()
                out_row = (b * CAMERAS + cam) * ROWS + h * HROWS
                wr_cps[p] = pltpu.async_copy(
                    buf_v.at[p], f_out.at[pl.ds(out_row, HROWS)], wsem)

    for p in range(2):
        if wr_cps[p] is not None:
            wr_cps[p].wait()


@jax.jit
def kernel(timestamps, keypoints, keypoint_confidence, trial_lengths, idx):
    nrow = NPANEL * PANEL
    kp_flat = keypoints.astype(jnp.float32).reshape(nrow, KP_D)
    conf_flat = keypoint_confidence.astype(jnp.float32).reshape(nrow, KPTS)
    ts_t = timestamps.astype(jnp.float32).T
    lens = trial_lengths.astype(jnp.int32)
    idx32 = idx.astype(jnp.int32)

    fused = pl.pallas_call(
        _build_body,
        grid=(NPANEL,),
        in_specs=[
            pl.BlockSpec((PANEL, KP_D), lambda p: (p, 0)),
            pl.BlockSpec((PANEL, KPTS), lambda p: (p, 0)),
            pl.BlockSpec((MAX_LEN, NUM_TRIALS), lambda p: (0, 0)),
        ],
        out_specs=pl.BlockSpec((PANEL, FUSE_D), lambda p: (p, 0)),
        out_shape=jax.ShapeDtypeStruct((nrow, FUSE_D), jnp.float32),
        compiler_params=pltpu.CompilerParams(
            dimension_semantics=("arbitrary",)),
    )(kp_flat, conf_flat, ts_t)

    mesh = plsc.VectorSubcoreMesh(core_axis_name="c", subcore_axis_name="s")
    run = pl.kernel(
        _sc_body,
        out_type=[
            jax.ShapeDtypeStruct((BATCH,), jnp.int32),
            jax.ShapeDtypeStruct((BATCH * CAMERAS * SAMPLE_LENGTH, FUSE_D),
                                 jnp.float32),
        ],
        mesh=mesh,
        compiler_params=pltpu.CompilerParams(use_tc_tiling_on_sc=False),
        scratch_types=[
            pltpu.VMEM((BATCH,), jnp.int32),
            pltpu.VMEM((NUM_TRIALS,), jnp.int32),
            pltpu.VMEM((CAMERAS * NCHUNK, CHUNK), jnp.int32),
            pltpu.VMEM((2, HROWS, FUSE_D), jnp.float32),
            pltpu.VMEM((BATCH,), jnp.int32),
            pltpu.SemaphoreType.DMA,
            pltpu.SemaphoreType.DMA,
        ],
    )
    ii, fo = run(fused, lens, idx32)

    brows = CAMERAS * SAMPLE_LENGTH
    kp_o, conf_o = pl.pallas_call(
        _split_body,
        grid=(BATCH,),
        in_specs=[pl.BlockSpec((brows, FUSE_D), lambda b: (b, 0))],
        out_specs=[
            pl.BlockSpec((brows, KP_D), lambda b: (b, 0)),
            pl.BlockSpec((brows, KPTS), lambda b: (b, 0)),
        ],
        out_shape=[
            jax.ShapeDtypeStruct((BATCH * brows, KP_D), jnp.float32),
            jax.ShapeDtypeStruct((BATCH * brows, KPTS), jnp.float32),
        ],
        compiler_params=pltpu.CompilerParams(
            dimension_semantics=("arbitrary",)),
    )(fo)

    kp_s = kp_o.reshape(BATCH, CAMERAS, SAMPLE_LENGTH, KPTS, 2)
    conf_s = conf_o.reshape(BATCH, CAMERAS, SAMPLE_LENGTH, KPTS)
    ts_s = fo.reshape(BATCH, CAMERAS, SAMPLE_LENGTH, FUSE_D)[:, 0, :,
                                                             KP_D + KPTS]
    return ((ii, ts_s), (kp_s, conf_s))


# speedup vs baseline: 19.1754x; 1.0750x over previous
import jax
import jax.numpy as jnp
from jax import lax
from jax.experimental import pallas as pl
from jax.experimental.pallas import tpu as pltpu
from jax.experimental.pallas import tpu_sc as plsc

NUM_TRIALS = 16
MAX_LEN = 4096
CAMERAS = 6
KPTS = 25
SAMPLE_LENGTH = 1024
BATCH = 64

NC = 2
NS = 16
NW = NC * NS
B_PER_W = BATCH // NW

KP_D = KPTS * 2
FUSE_D = 80
ROWS = SAMPLE_LENGTH
CHUNK = 128
NCHUNK = ROWS // CHUNK
HROWS = ROWS // 2
HCHUNK = NCHUNK // 2


PANEL = MAX_LEN
NPANEL = NUM_TRIALS * CAMERAS


def _build_body(kp_ref, conf_ref, ts_ref, out_ref):
    kp = kp_ref[...]
    conf = conf_ref[...]
    ii = pl.program_id(0) // CAMERAS
    lane = lax.broadcasted_iota(jnp.int32, (PANEL, NUM_TRIALS), 1)
    ts = jnp.sum(jnp.where(lane == ii, ts_ref[...], 0.0), axis=1,
                 keepdims=True)
    pad = jnp.zeros((PANEL, FUSE_D - KP_D - KPTS - 1), jnp.float32)
    out_ref[...] = jnp.concatenate([kp, conf, ts, pad], axis=1)


def _split_body(f_ref, kp_ref, conf_ref):
    f = f_ref[...]
    kp_t = jnp.transpose(f[:, :KP_D])
    kp_ref[...] = kp_t.reshape(1, 1, KPTS, 2, SAMPLE_LENGTH)
    conf_ref[...] = f[:, KP_D:KP_D + KPTS]


def _splat(vec, idxvec):
    dnums = lax.GatherDimensionNumbers(
        offset_dims=(), collapsed_slice_dims=(0,), start_index_map=(0,))
    return lax.gather(vec, idxvec.reshape(16, 1), dnums, (1,),
                      mode=lax.GatherScatterMode.PROMISE_IN_BOUNDS)


def _sc_body(fused_hbm, len_hbm, idx_hbm,
             ii_out, f_out,
             idx_v, len_v, gidx_v, buf_v, ii_buf, sem, wsem):
    c = lax.axis_index("c")
    s = lax.axis_index("s")
    wid = s * NC + c

    pltpu.sync_copy(idx_hbm, idx_v)
    pltpu.sync_copy(len_hbm, len_v)

    @pl.when(wid == 0)
    def _():
        for k in range(BATCH // 16):
            ii_buf[pl.ds(k * 16, 16)] = idx_v[pl.ds(k * 16, 16)] & (
                NUM_TRIALS - 1)
        pltpu.sync_copy(ii_buf, ii_out)

    len16 = len_v[...]
    iot = lax.iota(jnp.int32, 16)
    wr_cps = [None, None]

    for r in range(B_PER_W):
        b = wid * B_PER_W + r
        win = idx_v[pl.ds(((b >> 4) << 4), 16)]
        iv_v = _splat(win, iot * 0 + (b & 15))
        start_v = iv_v >> 4
        ii_v = iv_v & (NUM_TRIALS - 1)
        lv_v = _splat(len16, ii_v)
        nv_v = (lv_v + (SAMPLE_LENGTH - 1)) >> 10
        base_v = ii_v * (CAMERAS * MAX_LEN)

        for rw in range(NCHUNK):
            for k in range(CHUNK // 16):
                tv = iot + (rw * CHUNK + k * 16)
                vv = tv * nv_v + start_v
                sv = vv - jnp.where(vv >= lv_v, lv_v, 0)
                rb = base_v + sv
                for cam in range(CAMERAS):
                    gidx_v[cam * NCHUNK + rw, pl.ds(k * 16, 16)] = (
                        rb + cam * MAX_LEN)

        for cam in range(CAMERAS):
            for h in range(2):
                p = (cam * 2 + h) & 1
                cps = [
                    pltpu.async_copy(
                        fused_hbm.at[gidx_v.at[cam * NCHUNK + h * HCHUNK + j]],
                        buf_v.at[p, pl.ds(j * CHUNK, CHUNK)], sem)
                    for j in range(HCHUNK)
                ]
                for cp in cps:
                    cp.wait()
                if wr_cps[p] is not None:
                    wr_cps[p].wait()
                out_row = (b * CAMERAS + cam) * ROWS + h * HROWS
                wr_cps[p] = pltpu.async_copy(
                    buf_v.at[p], f_out.at[pl.ds(out_row, HROWS)], wsem)

    for p in range(2):
        if wr_cps[p] is not None:
            wr_cps[p].wait()


@jax.jit
def kernel(timestamps, keypoints, keypoint_confidence, trial_lengths, idx):
    nrow = NPANEL * PANEL
    kp_flat = keypoints.astype(jnp.float32).reshape(nrow, KP_D)
    conf_flat = keypoint_confidence.astype(jnp.float32).reshape(nrow, KPTS)
    ts_t = timestamps.astype(jnp.float32).T
    lens = trial_lengths.astype(jnp.int32)
    idx32 = idx.astype(jnp.int32)

    fused = pl.pallas_call(
        _build_body,
        grid=(NPANEL,),
        in_specs=[
            pl.BlockSpec((PANEL, KP_D), lambda p: (p, 0)),
            pl.BlockSpec((PANEL, KPTS), lambda p: (p, 0)),
            pl.BlockSpec((MAX_LEN, NUM_TRIALS), lambda p: (0, 0)),
        ],
        out_specs=pl.BlockSpec((PANEL, FUSE_D), lambda p: (p, 0)),
        out_shape=jax.ShapeDtypeStruct((nrow, FUSE_D), jnp.float32),
        compiler_params=pltpu.CompilerParams(
            dimension_semantics=("arbitrary",)),
    )(kp_flat, conf_flat, ts_t)

    mesh = plsc.VectorSubcoreMesh(core_axis_name="c", subcore_axis_name="s")
    run = pl.kernel(
        _sc_body,
        out_type=[
            jax.ShapeDtypeStruct((BATCH,), jnp.int32),
            jax.ShapeDtypeStruct((BATCH * CAMERAS * SAMPLE_LENGTH, FUSE_D),
                                 jnp.float32),
        ],
        mesh=mesh,
        compiler_params=pltpu.CompilerParams(use_tc_tiling_on_sc=False),
        scratch_types=[
            pltpu.VMEM((BATCH,), jnp.int32),
            pltpu.VMEM((NUM_TRIALS,), jnp.int32),
            pltpu.VMEM((CAMERAS * NCHUNK, CHUNK), jnp.int32),
            pltpu.VMEM((2, HROWS, FUSE_D), jnp.float32),
            pltpu.VMEM((BATCH,), jnp.int32),
            pltpu.SemaphoreType.DMA,
            pltpu.SemaphoreType.DMA,
        ],
    )
    ii, fo = run(fused, lens, idx32)

    kp_nat, conf_o = pl.pallas_call(
        _split_body,
        grid=(BATCH, CAMERAS),
        in_specs=[pl.BlockSpec((SAMPLE_LENGTH, FUSE_D),
                               lambda b, c: (b * CAMERAS + c, 0))],
        out_specs=[
            pl.BlockSpec((1, 1, KPTS, 2, SAMPLE_LENGTH),
                         lambda b, c: (b, c, 0, 0, 0)),
            pl.BlockSpec((SAMPLE_LENGTH, KPTS),
                         lambda b, c: (b * CAMERAS + c, 0)),
        ],
        out_shape=[
            jax.ShapeDtypeStruct((BATCH, CAMERAS, KPTS, 2, SAMPLE_LENGTH),
                                 jnp.float32),
            jax.ShapeDtypeStruct((BATCH * CAMERAS * SAMPLE_LENGTH, KPTS),
                                 jnp.float32),
        ],
        compiler_params=pltpu.CompilerParams(
            dimension_semantics=("arbitrary", "arbitrary")),
    )(fo)

    kp_s = jnp.transpose(kp_nat, (0, 1, 4, 2, 3))
    conf_s = conf_o.reshape(BATCH, CAMERAS, SAMPLE_LENGTH, KPTS)
    ts_s = fo.reshape(BATCH, CAMERAS, SAMPLE_LENGTH, FUSE_D)[:, 0, :,
                                                             KP_D + KPTS]
    return ((ii, ts_s), (kp_s, conf_s))
